# Initial kernel scaffold; baseline (speedup 1.0000x reference)
import jax
import jax.numpy as jnp
from jax.experimental import pallas as pl


def kernel(p, fc_pos_W, fc_pos_b, blk_fc0_W, blk_fc0_b, blk_fc1_W, blk_fc1_b, blk_sc_W, fc_c_W, fc_c_b):
    raise NotImplementedError("write your pallas kernel here")


# speedup vs baseline: 1.0000x; 1.0000x over previous
import jax
import jax.numpy as jnp
from jax.experimental import pallas as pl

RESO = 32
PAD = 0.1
H = 32
C_OUT = 32
N_BLOCKS = 5


def _resblock(x, W0, b0, W1, b1, Ws):
    net = jax.nn.relu(x) @ W0 + b0
    dx = jax.nn.relu(net) @ W1 + b1
    return x @ Ws + dx


def _coord_index(coords):
    p_nor = coords / (1 + PAD + 10e-4) + 0.5
    p_nor = jnp.clip(p_nor, 0.0, 1 - 10e-4)
    xg = (jax.lax.stop_gradient(p_nor) * RESO).astype(jnp.int32)
    return xg[:, :, 0] + RESO * (xg[:, :, 1] + RESO * xg[:, :, 2])


def _scatter_max(src, idx, dim_size):
    def one(s, i):
        out = jnp.full((s.shape[0], dim_size), -jnp.inf, dtype=s.dtype).at[:, i].max(s)
        return jnp.where(jnp.isneginf(out), 0.0, out)
    return jax.vmap(one)(src, idx)


def _scatter_mean(src, idx, dim_size):
    def one(s, i):
        ssum = jnp.zeros((s.shape[0], dim_size), s.dtype).at[:, i].add(s)
        cnt = jnp.zeros((dim_size,), s.dtype).at[i].add(1.0)
        return ssum / jnp.maximum(cnt, 1.0)
    return jax.vmap(one)(src, idx)


def kernel(p, fc_pos_W, fc_pos_b, blk_fc0_W, blk_fc0_b, blk_fc1_W, blk_fc1_b, blk_sc_W, fc_c_W, fc_c_b):
    B, N, _ = p.shape
    coords = p[:, :, :3]
    idx = _coord_index(coords)
    net = coords @ fc_pos_W + fc_pos_b
    net = _resblock(net, blk_fc0_W[0], blk_fc0_b[0], blk_fc1_W[0], blk_fc1_b[0], blk_sc_W[0])
    for i in range(1, N_BLOCKS):
        c_perm = jnp.swapaxes(net, 1, 2)
        fea = _scatter_max(c_perm, idx, RESO ** 3)
        idx_e = jnp.broadcast_to(idx[:, None, :], (B, fea.shape[1], N))
        pooled = jnp.take_along_axis(fea, idx_e, axis=2)
        pooled = jnp.swapaxes(pooled, 1, 2)
        net = jnp.concatenate([net, pooled], axis=2)
        net = _resblock(net, blk_fc0_W[i], blk_fc0_b[i], blk_fc1_W[i], blk_fc1_b[i], blk_sc_W[i])
    c = net @ fc_c_W + fc_c_b
    c_perm = jnp.swapaxes(c, 1, 2)
    fea_grid = _scatter_mean(c_perm, idx, RESO ** 3)
    return fea_grid.reshape(B, C_OUT, RESO, RESO, RESO)


# speedup vs baseline: 208.2343x; 208.2304x over previous
import functools

import jax
import jax.numpy as jnp
from jax import lax
from jax.experimental import pallas as pl
from jax.experimental.pallas import tpu as pltpu
from jax.experimental.pallas import tpu_sc as plsc

RESO = 32
PAD = 0.1
H = 32
C_OUT = 32
N_BLOCKS = 5
BLK = 10000

NC = 2
NS = 16
NW = NC * NS
L = 16
R3 = RESO ** 3
CPT = R3 // NW
NPERM = 102400
PCH = 512
POSMASK = 0x1FFFF


def _first_stage_body(p_ref, W_ref, b_ref, W0_ref, b0_ref, W1_ref, b1_ref, Ws_ref,
                      idx_ref, net_ref):
    coords = p_ref[0]
    p_nor = coords / (1 + PAD + 10e-4) + 0.5
    p_nor = jnp.clip(p_nor, 0.0, 1 - 10e-4)
    xg = (p_nor * RESO).astype(jnp.int32)
    idx_ref[0] = xg[:, 0:1] + RESO * (xg[:, 1:2] + RESO * xg[:, 2:3])
    x = coords @ W_ref[...] + b_ref[...][None, :]
    net = jax.nn.relu(x) @ W0_ref[...] + b0_ref[...][None, :]
    dx = jax.nn.relu(net) @ W1_ref[...] + b1_ref[...][None, :]
    net_ref[0] = x @ Ws_ref[...] + dx


def _mid_stage_body(net_in_ref, pooled_ref, W0_ref, b0_ref, W1_ref, b1_ref, Ws_ref,
                    net_out_ref):
    x = jnp.concatenate([net_in_ref[0], pooled_ref[0]], axis=1)
    net = jax.nn.relu(x) @ W0_ref[...] + b0_ref[...][None, :]
    dx = jax.nn.relu(net) @ W1_ref[...] + b1_ref[...][None, :]
    net_out_ref[0] = x @ Ws_ref[...] + dx


def _last_stage_body(net_in_ref, pooled_ref, W0_ref, b0_ref, W1_ref, b1_ref, Ws_ref,
                     Wc_ref, bc_ref, c_ref):
    x = jnp.concatenate([net_in_ref[0], pooled_ref[0]], axis=1)
    net = jax.nn.relu(x) @ W0_ref[...] + b0_ref[...][None, :]
    dx = jax.nn.relu(net) @ W1_ref[...] + b1_ref[...][None, :]
    net = x @ Ws_ref[...] + dx
    c_ref[0] = net @ Wc_ref[...] + bc_ref[...][None, :]


def _wspec(shape):
    return pl.BlockSpec(shape, lambda b, n: (0,) * len(shape))


def _tc_first_stage(p, fc_pos_W, fc_pos_b, W0, b0, W1, b1, Ws):
    B, N, _ = p.shape
    grid = (B, N // BLK)
    idx3, net = pl.pallas_call(
        _first_stage_body,
        grid=grid,
        in_specs=[
            pl.BlockSpec((1, BLK, 3), lambda b, n: (b, n, 0)),
            _wspec(fc_pos_W.shape), _wspec(fc_pos_b.shape),
            _wspec(W0.shape), _wspec(b0.shape),
            _wspec(W1.shape), _wspec(b1.shape), _wspec(Ws.shape),
        ],
        out_specs=[
            pl.BlockSpec((1, BLK, 1), lambda b, n: (b, n, 0)),
            pl.BlockSpec((1, BLK, H), lambda b, n: (b, n, 0)),
        ],
        out_shape=[
            jax.ShapeDtypeStruct((B, N, 1), jnp.int32),
            jax.ShapeDtypeStruct((B, N, H), jnp.float32),
        ],
    )(p, fc_pos_W, fc_pos_b, W0, b0, W1, b1, Ws)
    return idx3, net


def _tc_mid_stage(net, pooled_pad, W0, b0, W1, b1, Ws):
    B, N, _ = net.shape
    grid = (B, N // BLK)
    return pl.pallas_call(
        _mid_stage_body,
        grid=grid,
        in_specs=[
            pl.BlockSpec((1, BLK, H), lambda b, n: (b, n, 0)),
            pl.BlockSpec((1, BLK, H), lambda b, n: (b, n, 0)),
            _wspec(W0.shape), _wspec(b0.shape),
            _wspec(W1.shape), _wspec(b1.shape), _wspec(Ws.shape),
        ],
        out_specs=pl.BlockSpec((1, BLK, H), lambda b, n: (b, n, 0)),
        out_shape=jax.ShapeDtypeStruct((B, N, H), jnp.float32),
    )(net, pooled_pad, W0, b0, W1, b1, Ws)


def _tc_last_stage(net, pooled_pad, W0, b0, W1, b1, Ws, Wc, bc):
    B, N, _ = net.shape
    grid = (B, N // BLK)
    return pl.pallas_call(
        _last_stage_body,
        grid=grid,
        in_specs=[
            pl.BlockSpec((1, BLK, H), lambda b, n: (b, n, 0)),
            pl.BlockSpec((1, BLK, H), lambda b, n: (b, n, 0)),
            _wspec(W0.shape), _wspec(b0.shape),
            _wspec(W1.shape), _wspec(b1.shape), _wspec(Ws.shape),
            _wspec(Wc.shape), _wspec(bc.shape),
        ],
        out_specs=pl.BlockSpec((1, BLK, C_OUT), lambda b, n: (b, n, 0)),
        out_shape=jax.ShapeDtypeStruct((B, N, C_OUT), jnp.float32),
    )(net, pooled_pad, W0, b0, W1, b1, Ws, Wc, bc)


def _sc_mesh():
    return plsc.VectorSubcoreMesh(core_axis_name="c", subcore_axis_name="s")


_GDN = lax.GatherDimensionNumbers(offset_dims=(), collapsed_slice_dims=(0,),
                                  start_index_map=(0,))


def _lane_bcast(vec, i):
    idx = jnp.full((L, 1), i, jnp.int32)
    return lax.gather(vec, idx, _GDN, (1,),
                      mode=lax.GatherScatterMode.PROMISE_IN_BOUNDS)


def _worker_id():
    return lax.axis_index("s") * NC + lax.axis_index("c")


def _make_partition(B, N):
    CH = 10000
    NCH = N // CH
    STAGE = N + L
    FCH = 2048

    @functools.partial(
        pl.kernel,
        out_type=[
            jax.ShapeDtypeStruct((B, NPERM), jnp.int32),
            jax.ShapeDtypeStruct((B, NW, L), jnp.int32),
            jax.ShapeDtypeStruct((B, NW, L), jnp.int32),
        ],
        mesh=_sc_mesh(),
        compiler_params=pltpu.CompilerParams(use_tc_tiling_on_sc=False, needs_layout_passes=False),
        scratch_types=[
            pltpu.VMEM((CH,), jnp.int32),
            pltpu.VMEM((STAGE,), jnp.int32),
            pltpu.VMEM((L,), jnp.int32),
        ],
    )
    def part(idx_hbm, perm_hbm, starts_hbm, cnts_hbm, chunk_v, stage_v, meta_v):
        w = _worker_id()
        cell_lo = w * CPT
        iota = lax.iota(jnp.int32, L)

        @pl.loop(0, B)
        def _batch(b):
            def vbody(g, carry):
                lt, eq = carry
                v = chunk_v[pl.ds(g * L, L)]
                mlt = v < cell_lo
                meq = (v >= cell_lo) & (v < cell_lo + CPT)
                return (lt + plsc.all_reduce_population_count(mlt),
                        eq + plsc.all_reduce_population_count(meq))

            def cbody(c, carry):
                pltpu.sync_copy(idx_hbm.at[b, pl.ds(c * CH, CH)], chunk_v)
                return lax.fori_loop(0, CH // L, vbody, carry)

            z16 = jnp.zeros((L,), jnp.int32)
            lt, eq = lax.fori_loop(0, NCH, cbody, (z16, z16))
            prefix = jnp.max(lt)
            cnt = jnp.max(eq)
            start = pl.multiple_of(((prefix + 32 * w) + 15) & ~15, 16)

            meta_v[...] = jnp.full((L,), start, jnp.int32)
            pltpu.sync_copy(meta_v, starts_hbm.at[b, w])
            meta_v[...] = jnp.full((L,), cnt, jnp.int32)
            pltpu.sync_copy(meta_v, cnts_hbm.at[b, w])

            def cbody2(c, nst):
                pltpu.sync_copy(idx_hbm.at[b, pl.ds(c * CH, CH)], chunk_v)

                def vbody2(g, nst):
                    v = chunk_v[pl.ds(g * L, L)]
                    meq = (v >= cell_lo) & (v < cell_lo + CPT)
                    pos = c * CH + g * L + iota
                    packed = pos | ((v - cell_lo) << 17)
                    plsc.store_compressed(stage_v.at[pl.ds(nst, L)], packed,
                                          mask=meq)
                    return nst + jnp.max(plsc.all_reduce_population_count(meq))

                return lax.fori_loop(0, CH // L, vbody2, nst)

            nst = lax.fori_loop(0, NCH, cbody2, jnp.int32(0))

            nfull = nst // FCH

            @pl.loop(0, nfull)
            def _flush(k):
                pltpu.sync_copy(stage_v.at[pl.ds(k * FCH, FCH)],
                                perm_hbm.at[b, pl.ds(pl.multiple_of(start + k * FCH, 16), FCH)])

            off = nfull * FCH
            rem16 = ((nst - off) + 15) & ~15
            for sz in (1024, 512, 256, 128, 64, 32, 16):
                pred = (rem16 & sz) != 0

                def _mk(off_, sz_):
                    def _do():
                        pltpu.sync_copy(stage_v.at[pl.ds(off_, sz_)],
                                        perm_hbm.at[b, pl.ds(pl.multiple_of(start + off_, 16), sz_)])
                    return _do

                pl.when(pred)(_mk(off, sz))
                off = off + jnp.where(pred, sz, 0)

    return part


def _make_pool_max(B, N):
    @functools.partial(
        pl.kernel,
        out_type=jax.ShapeDtypeStruct((B, N + L, H), jnp.float32),
        mesh=_sc_mesh(),
        compiler_params=pltpu.CompilerParams(use_tc_tiling_on_sc=False, needs_layout_passes=False),
        scratch_types=[
            pltpu.VMEM((CPT * H,), jnp.float32),
            pltpu.VMEM((PCH,), jnp.int32),
            pltpu.VMEM((4, 128), jnp.int32),
            pltpu.VMEM((PCH, H), jnp.float32),
            pltpu.VMEM((PCH, H), jnp.float32),
            pltpu.VMEM((L,), jnp.int32),
            pltpu.SemaphoreType.DMA,
        ],
    )
    def poolmax(net_hbm, perm_hbm, starts_hbm, cnts_hbm, pooled_hbm,
                slab, pchunk, posb, rows, outr, meta_v, sem):
        w = _worker_id()
        iota = lax.iota(jnp.int32, L)

        @pl.loop(0, B)
        def _batch(b):
            pltpu.sync_copy(starts_hbm.at[b, w], meta_v)
            start = jnp.max(meta_v[...])
            pltpu.sync_copy(cnts_hbm.at[b, w], meta_v)
            cnt = jnp.max(meta_v[...])

            @pl.loop(0, CPT * H // L, unroll=8)
            def _init(i):
                slab[pl.ds(i * L, L)] = jnp.full((L,), -jnp.inf, jnp.float32)

            nchunks = (cnt + PCH - 1) // PCH

            @pl.loop(0, nchunks)
            def _scat(ci):
                base = pl.multiple_of(start + ci * PCH, 16)
                csize = jnp.minimum(PCH, cnt - ci * PCH)
                pltpu.sync_copy(perm_hbm.at[b, pl.ds(base, PCH)], pchunk)
                for k in range(4):
                    @pl.loop(0, 128 // L)
                    def _unpack(g2, k=k):
                        pk = pchunk[pl.ds((k * 8 + g2) * L, L)]
                        pos = jnp.minimum(pk & POSMASK, N - 1)
                        posb[k, pl.ds(g2 * L, L)] = pos

                    pltpu.async_copy(net_hbm.at[b].at[posb.at[k]],
                                     rows.at[pl.ds(k * 128, 128)], sem).wait()

                @pl.loop(0, (csize + L - 1) // L)
                def _rmw(g):
                    pk = pchunk[pl.ds(g * L, L)]
                    loc = lax.shift_right_logical(pk, 17) & (CPT - 1)
                    for i in range(L):
                        j = g * L + i
                        lb = _lane_bcast(loc, i)
                        a0 = lb * H + iota
                        jv = jnp.full((L,), j, jnp.int32)
                        vj = j < csize
                        r0 = plsc.load_gather(rows, [jv, iota])
                        r1 = plsc.load_gather(rows, [jv, iota + L])
                        r0 = jnp.where(vj, r0, -jnp.inf)
                        r1 = jnp.where(vj, r1, -jnp.inf)
                        c0 = plsc.load_gather(slab, [a0])
                        c1 = plsc.load_gather(slab, [a0 + L])
                        plsc.store_scatter(slab, [a0], jnp.maximum(c0, r0))
                        plsc.store_scatter(slab, [a0 + L], jnp.maximum(c1, r1))

            @pl.loop(0, nchunks)
            def _gat(ci):
                base = pl.multiple_of(start + ci * PCH, 16)
                csize = jnp.minimum(PCH, cnt - ci * PCH)
                pltpu.sync_copy(perm_hbm.at[b, pl.ds(base, PCH)], pchunk)

                @pl.loop(0, PCH // L)
                def _read(g):
                    pk = pchunk[pl.ds(g * L, L)]
                    loc = lax.shift_right_logical(pk, 17) & (CPT - 1)
                    for i in range(L):
                        lb = _lane_bcast(loc, i)
                        a0 = lb * H + iota
                        jv = jnp.full((L,), g * L + i, jnp.int32)
                        v0 = plsc.load_gather(slab, [a0])
                        v1 = plsc.load_gather(slab, [a0 + L])
                        plsc.store_scatter(outr, [jv, iota], v0)
                        plsc.store_scatter(outr, [jv, iota + L], v1)

                for k in range(4):
                    @pl.loop(0, 128 // L)
                    def _unpack(g2, k=k):
                        g = k * 8 + g2
                        pk = pchunk[pl.ds(g * L, L)]
                        valid = (g * L + iota) < csize
                        pos = jnp.where(valid, jnp.minimum(pk & POSMASK, N - 1),
                                        N)
                        posb[k, pl.ds(g2 * L, L)] = pos

                    pltpu.async_copy(outr.at[pl.ds(k * 128, 128)],
                                     pooled_hbm.at[b].at[posb.at[k]], sem).wait()

    return poolmax


def _make_scatter_mean(B, N):
    @functools.partial(
        pl.kernel,
        out_type=jax.ShapeDtypeStruct((B, C_OUT, R3), jnp.float32),
        mesh=_sc_mesh(),
        compiler_params=pltpu.CompilerParams(use_tc_tiling_on_sc=False, needs_layout_passes=False),
        scratch_types=[
            pltpu.VMEM((CPT * H,), jnp.float32),
            pltpu.VMEM((CPT,), jnp.float32),
            pltpu.VMEM((CPT,), jnp.float32),
            pltpu.VMEM((PCH,), jnp.int32),
            pltpu.VMEM((4, 128), jnp.int32),
            pltpu.VMEM((PCH, H), jnp.float32),
            pltpu.VMEM((L,), jnp.int32),
            pltpu.SemaphoreType.DMA,
        ],
    )
    def smean(c_hbm, perm_hbm, starts_hbm, cnts_hbm, grid_hbm,
              slab, cntb, colbuf, pchunk, posb, rows, meta_v, sem):
        w = _worker_id()
        iota = lax.iota(jnp.int32, L)
        lane0 = iota == 0

        @pl.loop(0, B)
        def _batch(b):
            pltpu.sync_copy(starts_hbm.at[b, w], meta_v)
            start = jnp.max(meta_v[...])
            pltpu.sync_copy(cnts_hbm.at[b, w], meta_v)
            cnt = jnp.max(meta_v[...])

            @pl.loop(0, CPT * H // L, unroll=8)
            def _init(i):
                slab[pl.ds(i * L, L)] = jnp.zeros((L,), jnp.float32)

            @pl.loop(0, CPT // L, unroll=8)
            def _init2(i):
                cntb[pl.ds(i * L, L)] = jnp.zeros((L,), jnp.float32)

            nchunks = (cnt + PCH - 1) // PCH

            @pl.loop(0, nchunks)
            def _scat(ci):
                base = pl.multiple_of(start + ci * PCH, 16)
                csize = jnp.minimum(PCH, cnt - ci * PCH)
                pltpu.sync_copy(perm_hbm.at[b, pl.ds(base, PCH)], pchunk)
                for k in range(4):
                    @pl.loop(0, 128 // L)
                    def _unpack(g2, k=k):
                        pk = pchunk[pl.ds((k * 8 + g2) * L, L)]
                        pos = jnp.minimum(pk & POSMASK, N - 1)
                        posb[k, pl.ds(g2 * L, L)] = pos

                    pltpu.async_copy(c_hbm.at[b].at[posb.at[k]],
                                     rows.at[pl.ds(k * 128, 128)], sem).wait()

                @pl.loop(0, (csize + L - 1) // L)
                def _rmw(g):
                    pk = pchunk[pl.ds(g * L, L)]
                    loc = lax.shift_right_logical(pk, 17) & (CPT - 1)
                    for i in range(L):
                        j = g * L + i
                        lb = _lane_bcast(loc, i)
                        a0 = lb * H + iota
                        jv = jnp.full((L,), j, jnp.int32)
                        vj = j < csize
                        r0 = plsc.load_gather(rows, [jv, iota])
                        r1 = plsc.load_gather(rows, [jv, iota + L])
                        r0 = jnp.where(vj, r0, 0.0)
                        r1 = jnp.where(vj, r1, 0.0)
                        c0 = plsc.load_gather(slab, [a0])
                        c1 = plsc.load_gather(slab, [a0 + L])
                        plsc.store_scatter(slab, [a0], c0 + r0)
                        plsc.store_scatter(slab, [a0 + L], c1 + r1)
                        cc = plsc.load_gather(cntb, [lb])
                        cc = cc + jnp.where(vj, 1.0, 0.0)
                        plsc.store_scatter(cntb, [lb], cc, mask=lane0)

            for ch in range(H):
                @pl.loop(0, CPT // L)
                def _wout(gg, ch=ch):
                    av = (gg * L + iota) * H + ch
                    sv = plsc.load_gather(slab, [av])
                    cv = cntb[pl.ds(gg * L, L)]
                    colbuf[pl.ds(gg * L, L)] = sv / jnp.maximum(cv, 1.0)

                pltpu.sync_copy(colbuf, grid_hbm.at[b, ch, pl.ds(w * CPT, CPT)])

    return smean


def kernel(p, fc_pos_W, fc_pos_b, blk_fc0_W, blk_fc0_b, blk_fc1_W, blk_fc1_b, blk_sc_W, fc_c_W, fc_c_b):
    B, N, _ = p.shape
    idx3, net = _tc_first_stage(p, fc_pos_W, fc_pos_b,
                                blk_fc0_W[0], blk_fc0_b[0],
                                blk_fc1_W[0], blk_fc1_b[0], blk_sc_W[0])
    idx = idx3.reshape(B, N)
    part = _make_partition(B, N)
    perm, starts, cnts = part(idx)
    poolmax = _make_pool_max(B, N)
    for i in range(1, N_BLOCKS - 1):
        pooled_pad = poolmax(net, perm, starts, cnts)
        net = _tc_mid_stage(net, pooled_pad, blk_fc0_W[i], blk_fc0_b[i],
                            blk_fc1_W[i], blk_fc1_b[i], blk_sc_W[i])
    pooled_pad = poolmax(net, perm, starts, cnts)
    c = _tc_last_stage(net, pooled_pad, blk_fc0_W[4], blk_fc0_b[4],
                       blk_fc1_W[4], blk_fc1_b[4], blk_sc_W[4], fc_c_W, fc_c_b)
    smean = _make_scatter_mean(B, N)
    grid = smean(c, perm, starts, cnts)
    return grid.reshape(B, C_OUT, RESO, RESO, RESO)
